# Initial kernel scaffold; baseline (speedup 1.0000x reference)
import functools

import jax
import jax.numpy as jnp
from jax.experimental import pallas as pl
from jax.experimental.pallas import tpu as pltpu

T, D, E, F, FS = 2048, 1024, 64, 512, 512


def _moe_step(x_ref, wr_ref, wg_ref, wu_ref, wd_ref, wgs_ref, wus_ref,
              wds_ref, wsg_ref, out_ref, xbf_ref, i1_ref, i2_ref, w1_ref,
              w2_ref):
    e = pl.program_id(0)

    @pl.when(e == 0)
    def _init():
        x = x_ref[...]
        xbf_ref[...] = x.astype(jnp.bfloat16)
        logits = jnp.dot(x, wr_ref[...], preferred_element_type=jnp.float32)
        m = jnp.max(logits, axis=-1, keepdims=True)
        ex = jnp.exp(logits - m)
        p = ex / jnp.sum(ex, axis=-1, keepdims=True)
        cols = jax.lax.broadcasted_iota(jnp.int32, p.shape, 1)
        m1 = jnp.max(p, axis=-1, keepdims=True)
        i1 = jnp.min(jnp.where(p == m1, cols, E), axis=-1, keepdims=True)
        pm = jnp.where(cols == i1, -1.0, p)
        m2 = jnp.max(pm, axis=-1, keepdims=True)
        i2 = jnp.min(jnp.where(pm == m2, cols, E), axis=-1, keepdims=True)
        s = m1 + m2
        i1_ref[...] = i1
        i2_ref[...] = i2
        w1_ref[...] = m1 / s
        w2_ref[...] = m2 / s
        xb = x.astype(jnp.bfloat16)
        g = jnp.dot(xb, wgs_ref[...].astype(jnp.bfloat16),
                    preferred_element_type=jnp.float32)
        u = jnp.dot(xb, wus_ref[...].astype(jnp.bfloat16),
                    preferred_element_type=jnp.float32)
        h = (jax.nn.silu(g) * u).astype(jnp.bfloat16)
        sh = jnp.dot(h, wds_ref[...].astype(jnp.bfloat16),
                     preferred_element_type=jnp.float32)
        sg = jax.nn.sigmoid(jnp.dot(x, wsg_ref[...],
                                    preferred_element_type=jnp.float32))
        out_ref[...] = sg * sh

    xb = xbf_ref[...]
    g = jnp.dot(xb, wg_ref[0].astype(jnp.bfloat16),
                preferred_element_type=jnp.float32)
    u = jnp.dot(xb, wu_ref[0].astype(jnp.bfloat16),
                preferred_element_type=jnp.float32)
    h = (jax.nn.silu(g) * u).astype(jnp.bfloat16)
    hd = jnp.dot(h, wd_ref[0].astype(jnp.bfloat16),
                 preferred_element_type=jnp.float32)
    wcol = (jnp.where(i1_ref[...] == e, w1_ref[...], 0.0)
            + jnp.where(i2_ref[...] == e, w2_ref[...], 0.0))
    out_ref[...] += hd * wcol


def _moe_pallas(x, Wr, Wg_e, Wu_e, Wd_e, Wg_s, Wu_s, Wd_s, Wsg):
    return pl.pallas_call(
        _moe_step,
        grid=(E,),
        in_specs=[
            pl.BlockSpec((T, D), lambda e: (0, 0)),
            pl.BlockSpec((D, E), lambda e: (0, 0)),
            pl.BlockSpec((1, D, F), lambda e: (e, 0, 0)),
            pl.BlockSpec((1, D, F), lambda e: (e, 0, 0)),
            pl.BlockSpec((1, F, D), lambda e: (e, 0, 0)),
            pl.BlockSpec((D, FS), lambda e: (0, 0)),
            pl.BlockSpec((D, FS), lambda e: (0, 0)),
            pl.BlockSpec((FS, D), lambda e: (0, 0)),
            pl.BlockSpec((D, 1), lambda e: (0, 0)),
        ],
        out_specs=pl.BlockSpec((T, D), lambda e: (0, 0)),
        out_shape=jax.ShapeDtypeStruct((T, D), jnp.float32),
        scratch_shapes=[
            pltpu.VMEM((T, D), jnp.bfloat16),
            pltpu.VMEM((T, 1), jnp.int32),
            pltpu.VMEM((T, 1), jnp.int32),
            pltpu.VMEM((T, 1), jnp.float32),
            pltpu.VMEM((T, 1), jnp.float32),
        ],
        compiler_params=pltpu.CompilerParams(
            dimension_semantics=("arbitrary",),
        ),
    )(x, Wr, Wg_e, Wu_e, Wd_e, Wg_s, Wu_s, Wd_s, Wsg)


def kernel(hidden_states, Wr, Wg_e, Wu_e, Wd_e, Wg_s, Wu_s, Wd_s, Wsg):
    x = hidden_states.reshape(-1, hidden_states.shape[-1])
    out = _moe_pallas(x, Wr, Wg_e, Wu_e, Wd_e, Wg_s, Wu_s, Wd_s, Wsg)
    return out.reshape(hidden_states.shape)


# speedup vs baseline: 3.4491x; 3.4491x over previous
import functools

import jax
import jax.numpy as jnp
from jax.experimental import pallas as pl
from jax.experimental.pallas import tpu as pltpu

T, D, E, F, FS = 2048, 1024, 64, 512, 512
TT = 512
NCH = T // TT


def _moe_step(x_ref, wr_ref, wg_ref, wu_ref, wd_ref, wgs_ref, wus_ref,
              wds_ref, wsg_ref, out_ref, xbf_ref, i12_ref, w12_ref):
    e = pl.program_id(0)

    @pl.when(e == 0)
    def _init():
        for c in range(NCH):
            sl = pl.ds(c * TT, TT)
            x = x_ref[sl, :]
            xb = x.astype(jnp.bfloat16)
            xbf_ref[sl, :] = xb
            logits = jnp.dot(x, wr_ref[...], preferred_element_type=jnp.float32)
            m = jnp.max(logits, axis=-1, keepdims=True)
            ex = jnp.exp(logits - m)
            p = ex / jnp.sum(ex, axis=-1, keepdims=True)
            cols = jax.lax.broadcasted_iota(jnp.int32, p.shape, 1)
            m1 = jnp.max(p, axis=-1, keepdims=True)
            i1 = jnp.min(jnp.where(p == m1, cols, E), axis=-1, keepdims=True)
            pm = jnp.where(cols == i1, -1.0, p)
            m2 = jnp.max(pm, axis=-1, keepdims=True)
            i2 = jnp.min(jnp.where(pm == m2, cols, E), axis=-1, keepdims=True)
            s = m1 + m2
            i12_ref[sl, :] = jnp.concatenate([i1, i2], axis=1)
            w12_ref[sl, :] = jnp.concatenate([m1 / s, m2 / s], axis=1)
            g = jnp.dot(xb, wgs_ref[...].astype(jnp.bfloat16),
                        preferred_element_type=jnp.float32)
            u = jnp.dot(xb, wus_ref[...].astype(jnp.bfloat16),
                        preferred_element_type=jnp.float32)
            h = (jax.nn.silu(g) * u).astype(jnp.bfloat16)
            sh = jnp.dot(h, wds_ref[...].astype(jnp.bfloat16),
                         preferred_element_type=jnp.float32)
            sg = jax.nn.sigmoid(jnp.dot(x, wsg_ref[...],
                                        preferred_element_type=jnp.float32))
            out_ref[sl, :] = sg * sh

    wg = wg_ref[0].astype(jnp.bfloat16)
    wu = wu_ref[0].astype(jnp.bfloat16)
    wd = wd_ref[0].astype(jnp.bfloat16)
    for c in range(NCH):
        sl = pl.ds(c * TT, TT)
        xb = xbf_ref[sl, :]
        g = jnp.dot(xb, wg, preferred_element_type=jnp.float32)
        u = jnp.dot(xb, wu, preferred_element_type=jnp.float32)
        h = (jax.nn.silu(g) * u).astype(jnp.bfloat16)
        hd = jnp.dot(h, wd, preferred_element_type=jnp.float32)
        wcol = (jnp.where(i12_ref[sl, 0:1] == e, w12_ref[sl, 0:1], 0.0)
                + jnp.where(i12_ref[sl, 1:2] == e, w12_ref[sl, 1:2], 0.0))
        out_ref[sl, :] += hd * wcol


def _moe_pallas(x, Wr, Wg_e, Wu_e, Wd_e, Wg_s, Wu_s, Wd_s, Wsg):
    return pl.pallas_call(
        _moe_step,
        grid=(E,),
        in_specs=[
            pl.BlockSpec((T, D), lambda e: (0, 0)),
            pl.BlockSpec((D, E), lambda e: (0, 0)),
            pl.BlockSpec((1, D, F), lambda e: (e, 0, 0)),
            pl.BlockSpec((1, D, F), lambda e: (e, 0, 0)),
            pl.BlockSpec((1, F, D), lambda e: (e, 0, 0)),
            pl.BlockSpec((D, FS), lambda e: (0, 0)),
            pl.BlockSpec((D, FS), lambda e: (0, 0)),
            pl.BlockSpec((FS, D), lambda e: (0, 0)),
            pl.BlockSpec((D, 1), lambda e: (0, 0)),
        ],
        out_specs=pl.BlockSpec((T, D), lambda e: (0, 0)),
        out_shape=jax.ShapeDtypeStruct((T, D), jnp.float32),
        scratch_shapes=[
            pltpu.VMEM((T, D), jnp.bfloat16),
            pltpu.VMEM((T, 2), jnp.int32),
            pltpu.VMEM((T, 2), jnp.float32),
        ],
        compiler_params=pltpu.CompilerParams(
            dimension_semantics=("arbitrary",),
        ),
    )(x, Wr, Wg_e, Wu_e, Wd_e, Wg_s, Wu_s, Wd_s, Wsg)


def kernel(hidden_states, Wr, Wg_e, Wu_e, Wd_e, Wg_s, Wu_s, Wd_s, Wsg):
    x = hidden_states.reshape(-1, hidden_states.shape[-1])
    out = _moe_pallas(x, Wr, Wg_e, Wu_e, Wd_e, Wg_s, Wu_s, Wd_s, Wsg)
    return out.reshape(hidden_states.shape)


# speedup vs baseline: 3.8268x; 1.1095x over previous
import functools

import jax
import jax.numpy as jnp
from jax.experimental import pallas as pl
from jax.experimental.pallas import tpu as pltpu

T, D, E, F, FS = 2048, 1024, 64, 512, 512
TT = 512
NCH = T // TT
BT = 64
S = 8192
G = S // BT


def _router_shared_step(x_ref, wr_ref, wgs_ref, wus_ref, wds_ref, wsg_ref,
                        out_ref, xbf_ref, topi_ref, topw_ref):
    for c in range(NCH):
        sl = pl.ds(c * TT, TT)
        x = x_ref[sl, :]
        xb = x.astype(jnp.bfloat16)
        xbf_ref[sl, :] = xb
        logits = jnp.dot(x, wr_ref[...], preferred_element_type=jnp.float32)
        m = jnp.max(logits, axis=-1, keepdims=True)
        ex = jnp.exp(logits - m)
        p = ex / jnp.sum(ex, axis=-1, keepdims=True)
        cols = jax.lax.broadcasted_iota(jnp.int32, p.shape, 1)
        m1 = jnp.max(p, axis=-1, keepdims=True)
        i1 = jnp.min(jnp.where(p == m1, cols, E), axis=-1, keepdims=True)
        pm = jnp.where(cols == i1, -1.0, p)
        m2 = jnp.max(pm, axis=-1, keepdims=True)
        i2 = jnp.min(jnp.where(pm == m2, cols, E), axis=-1, keepdims=True)
        s = m1 + m2
        topi_ref[sl, :] = jnp.concatenate([i1, i2], axis=1)
        topw_ref[sl, :] = jnp.concatenate([m1 / s, m2 / s], axis=1)
        g = jnp.dot(xb, wgs_ref[...].astype(jnp.bfloat16),
                    preferred_element_type=jnp.float32)
        u = jnp.dot(xb, wus_ref[...].astype(jnp.bfloat16),
                    preferred_element_type=jnp.float32)
        h = (jax.nn.silu(g) * u).astype(jnp.bfloat16)
        sh = jnp.dot(h, wds_ref[...].astype(jnp.bfloat16),
                     preferred_element_type=jnp.float32)
        sg = jax.nn.sigmoid(jnp.dot(x, wsg_ref[...],
                                    preferred_element_type=jnp.float32))
        out_ref[sl, :] = sg * sh


def _router_shared(x, Wr, Wg_s, Wu_s, Wd_s, Wsg):
    return pl.pallas_call(
        _router_shared_step,
        grid=(1,),
        in_specs=[
            pl.BlockSpec((T, D), lambda i: (0, 0)),
            pl.BlockSpec((D, E), lambda i: (0, 0)),
            pl.BlockSpec((D, FS), lambda i: (0, 0)),
            pl.BlockSpec((D, FS), lambda i: (0, 0)),
            pl.BlockSpec((FS, D), lambda i: (0, 0)),
            pl.BlockSpec((D, 1), lambda i: (0, 0)),
        ],
        out_specs=[
            pl.BlockSpec((T, D), lambda i: (0, 0)),
            pl.BlockSpec((T, D), lambda i: (0, 0)),
            pl.BlockSpec((T, 2), lambda i: (0, 0)),
            pl.BlockSpec((T, 2), lambda i: (0, 0)),
        ],
        out_shape=[
            jax.ShapeDtypeStruct((T, D), jnp.float32),
            jax.ShapeDtypeStruct((T, D), jnp.bfloat16),
            jax.ShapeDtypeStruct((T, 2), jnp.int32),
            jax.ShapeDtypeStruct((T, 2), jnp.float32),
        ],
    )(x, Wr, Wg_s, Wu_s, Wd_s, Wsg)


def _dispatch_meta(topi, topw):
    eflat = topi.reshape(-1)
    wflat = topw.reshape(-1)
    onehot = (eflat[:, None] == jnp.arange(E, dtype=jnp.int32)[None, :])
    csum = jnp.cumsum(onehot.astype(jnp.int32), axis=0)
    rank = jnp.take_along_axis(csum, eflat[:, None], axis=1)[:, 0] - 1
    counts = csum[-1]
    ntiles_e = (counts + BT - 1) // BT
    tile_off = jnp.cumsum(ntiles_e) - ntiles_e
    n_tiles = jnp.sum(ntiles_e).astype(jnp.int32)
    pos = tile_off[eflat] * BT + rank
    sorted_tok = jnp.zeros((S,), jnp.int32).at[pos].set(
        jnp.arange(2 * T, dtype=jnp.int32) // 2)
    sorted_w = jnp.zeros((S,), jnp.float32).at[pos].set(wflat)
    g_ar = jnp.arange(G, dtype=jnp.int32)
    te_full = jnp.searchsorted(tile_off + ntiles_e, g_ar, side='right'
                               ).astype(jnp.int32)
    e_last = te_full[jnp.maximum(n_tiles - 1, 0)]
    tile_eid = jnp.where(g_ar < n_tiles, te_full, e_last)
    inv = pos.reshape(T, 2)
    return sorted_tok, sorted_w, tile_eid, n_tiles, inv


def _ggemm_step(teid_ref, nt_ref, tok_ref, sw_ref, xbf_ref, wg_ref, wu_ref,
                wd_ref, hg_ref):
    g = pl.program_id(0)

    @pl.when(g < nt_ref[0])
    def _():
        onehot = (jax.lax.broadcasted_iota(jnp.int32, (BT, T), 1)
                  == tok_ref[...]).astype(jnp.bfloat16)
        xg = jnp.dot(onehot, xbf_ref[...],
                     preferred_element_type=jnp.float32).astype(jnp.bfloat16)
        gg = jnp.dot(xg, wg_ref[0].astype(jnp.bfloat16),
                     preferred_element_type=jnp.float32)
        uu = jnp.dot(xg, wu_ref[0].astype(jnp.bfloat16),
                     preferred_element_type=jnp.float32)
        h = (jax.nn.silu(gg) * uu).astype(jnp.bfloat16)
        hd = jnp.dot(h, wd_ref[0].astype(jnp.bfloat16),
                     preferred_element_type=jnp.float32)
        hg_ref[...] = hd * sw_ref[...]


def _grouped_gemm(tile_eid, n_tiles, sorted_tok, sorted_w, xbf,
                  Wg_e, Wu_e, Wd_e):
    grid_spec = pltpu.PrefetchScalarGridSpec(
        num_scalar_prefetch=2,
        grid=(G,),
        in_specs=[
            pl.BlockSpec((BT, 1), lambda g, teid, nt: (g, 0)),
            pl.BlockSpec((BT, 1), lambda g, teid, nt: (g, 0)),
            pl.BlockSpec((T, D), lambda g, teid, nt: (0, 0)),
            pl.BlockSpec((1, D, F), lambda g, teid, nt: (teid[g], 0, 0)),
            pl.BlockSpec((1, D, F), lambda g, teid, nt: (teid[g], 0, 0)),
            pl.BlockSpec((1, F, D), lambda g, teid, nt: (teid[g], 0, 0)),
        ],
        out_specs=pl.BlockSpec((BT, D), lambda g, teid, nt: (g, 0)),
    )
    return pl.pallas_call(
        _ggemm_step,
        grid_spec=grid_spec,
        out_shape=jax.ShapeDtypeStruct((S, D), jnp.float32),
        compiler_params=pltpu.CompilerParams(
            dimension_semantics=("arbitrary",),
        ),
    )(tile_eid, n_tiles.reshape(1), sorted_tok.reshape(S, 1),
      sorted_w.reshape(S, 1), xbf, Wg_e, Wu_e, Wd_e)


def kernel(hidden_states, Wr, Wg_e, Wu_e, Wd_e, Wg_s, Wu_s, Wd_s, Wsg):
    x = hidden_states.reshape(-1, hidden_states.shape[-1])
    out_init, xbf, topi, topw = _router_shared(x, Wr, Wg_s, Wu_s, Wd_s, Wsg)
    sorted_tok, sorted_w, tile_eid, n_tiles, inv = _dispatch_meta(topi, topw)
    hg = _grouped_gemm(tile_eid, n_tiles, sorted_tok, sorted_w, xbf,
                       Wg_e, Wu_e, Wd_e)
    out = out_init + jnp.take(hg, inv[:, 0], axis=0) \
        + jnp.take(hg, inv[:, 1], axis=0)
    return out.reshape(hidden_states.shape)
